# baseline (speedup 1.0000x reference)
import functools

import jax
import jax.numpy as jnp
from jax import lax
from jax.experimental import pallas as pl
from jax.experimental.pallas import tpu as pltpu
from jax.experimental.pallas import tpu_sc as plsc

N = 10000
E = 320000
D = 128
PRED = 24
COUT = 7

NC = 2
NS = 16
NW = NC * NS
EPW = E // NW
CH = 80
NCHUNK = EPW // CH
DH = D // NC
EPS = E // NS
NCH_S = EPS // CH
NPAD = 10240
RPS = NPAD // NS
ZR = 128
CL = 16

_MESH = plsc.VectorSubcoreMesh(
    core_axis_name="c", subcore_axis_name="s", num_cores=NC, num_subcores=NS)
_SC_PARAMS = pltpu.CompilerParams(use_tc_tiling_on_sc=False)


def _make_sc_gather(ne, ch):
    epw = ne // NW
    nch = epw // ch
    assert epw % ch == 0 and ch % 8 == 0 and ch <= 128

    def body(h_hbm, s_hbm, r_hbm, g_hbm, idx_s, idx_r, rows_s, rows_r,
             rows_o, h_sh, *sems):
        sem_gs, sem_gr = sems[0:2], sems[2:4]
        sem_ws = sems[4:6]
        c = lax.axis_index("c")
        s = lax.axis_index("s")
        wid = s * NC + c
        base = wid * epw

        pltpu.sync_copy(s_hbm.at[pl.ds(wid * nch, nch)], idx_s)
        pltpu.sync_copy(r_hbm.at[pl.ds(wid * nch, nch)], idx_r)

        nps = N // NS
        pltpu.sync_copy(h_hbm.at[pl.ds(s * nps, nps)],
                        h_sh.at[pl.ds(s * nps, nps)])
        plsc.subcore_barrier()

        def start_gather(j, p):
            pltpu.async_copy(h_sh.at[idx_s.at[j]], rows_s.at[p], sem_gs[p])
            pltpu.async_copy(h_sh.at[idx_r.at[j]], rows_r.at[p], sem_gr[p])

        def wait_gather(j, p):
            pltpu.make_async_copy(h_sh.at[idx_s.at[j]], rows_s.at[p],
                                  sem_gs[p]).wait()
            pltpu.make_async_copy(h_sh.at[idx_r.at[j]], rows_r.at[p],
                                  sem_gr[p]).wait()

        def addrows(p):
            @plsc.parallel_loop(0, ch, unroll=4)
            def _(i):
                for k in range(D // 16):
                    sl = pl.ds(k * 16, 16)
                    rows_o[p, i, sl] = (rows_s[p, i, sl]
                                        + rows_r[p, i, sl]).astype(
                                            jnp.bfloat16)

        def start_wb(j, p):
            off = base + j * ch
            pltpu.async_copy(rows_o.at[p], g_hbm.at[pl.ds(off, ch)],
                             sem_ws[p])

        def wait_wb(p):
            pltpu.make_async_copy(rows_o.at[p], g_hbm.at[pl.ds(0, ch)],
                                  sem_ws[p]).wait()

        def pair(jj, carry):
            j0 = jj * 2
            for p in range(2):
                start_gather(j0 + p, p)
            for p in range(2):
                j = j0 + p
                wait_gather(j, p)
                addrows(p)
                start_wb(j, p)
            for p in range(2):
                wait_wb(p)
            return carry

        lax.fori_loop(0, nch // 2, pair, 0)
        if nch % 2:
            start_gather(nch - 1, 0)
            wait_gather(nch - 1, 0)
            addrows(0)
            start_wb(nch - 1, 0)
            wait_wb(0)

    k = pl.kernel(
        body,
        out_type=jax.ShapeDtypeStruct((ne, D), jnp.bfloat16),
        mesh=_MESH,
        compiler_params=_SC_PARAMS,
        scratch_types=[
            pltpu.VMEM((nch, ch), jnp.int32),
            pltpu.VMEM((nch, ch), jnp.int32),
            pltpu.VMEM((2, ch, D), jnp.float32),
            pltpu.VMEM((2, ch, D), jnp.float32),
            pltpu.VMEM((2, ch, D), jnp.bfloat16),
            pltpu.VMEM_SHARED((N, D), jnp.float32),
        ] + [pltpu.SemaphoreType.DMA] * 6,
    )

    def call(h, s, r):
        return k(h, s.reshape(-1, ch), r.reshape(-1, ch))

    return call


def _make_sc_scatter(with_counts, ne, ch, cnt_core=0):
    eps = ne // NS
    nchs = eps // ch
    assert eps % ch == 0 and nchs % 2 == 0 and ch % 8 == 0 and ch <= 128
    out_type = [jax.ShapeDtypeStruct((NPAD, D), jnp.float32)]
    scratch = [
        pltpu.VMEM((2, ch), jnp.int32),
        pltpu.VMEM((2, ch, DH), jnp.float32),
        pltpu.VMEM((ZR, DH), jnp.float32),
        pltpu.VMEM_SHARED((NPAD, DH), jnp.float32),
    ] + [pltpu.SemaphoreType.DMA] * 8
    if with_counts:
        out_type.append(jax.ShapeDtypeStruct((NPAD, CL), jnp.float32))
        scratch += [
            pltpu.VMEM((ch, CL), jnp.float32),
            pltpu.VMEM((RPS, CL), jnp.float32),
            pltpu.VMEM_SHARED((NPAD, CL), jnp.float32),
        ]

    def body(e_hbm, r_hbm, *refs):
        if with_counts:
            (part_hbm, cnt_hbm, idx, rows, zbuf, acc_sh,
             sli0, sli1, slr0, slr1, ssc0, ssc1, scn0, scn1,
             ones, cbuf, cnt_sh) = refs
        else:
            (part_hbm, idx, rows, zbuf, acc_sh,
             sli0, sli1, slr0, slr1, ssc0, ssc1, scn0, scn1) = refs
        sem_li, sem_lr = (sli0, sli1), (slr0, slr1)
        sem_sc, sem_cn = (ssc0, ssc1), (scn0, scn1)
        c = lax.axis_index("c")
        s = lax.axis_index("s")
        base = s * eps
        lane0 = c * DH
        zero16 = jnp.zeros((16,), jnp.float32)
        one16 = jnp.ones((16,), jnp.float32)

        def zrow(i, carry):
            for k in range(DH // 16):
                zbuf[i, pl.ds(k * 16, 16)] = zero16
            return carry
        lax.fori_loop(0, ZR, zrow, 0)
        if with_counts:
            def crow(i, carry):
                cbuf[i, pl.ds(0, 16)] = zero16
                return carry
            lax.fori_loop(0, RPS, crow, 0)

            def orow(i, carry):
                ones[i, pl.ds(0, 16)] = one16
                return carry
            lax.fori_loop(0, ch, orow, 0)

        for t in range(RPS // ZR):
            pltpu.sync_copy(zbuf, acc_sh.at[pl.ds(s * RPS + t * ZR, ZR)])
        if with_counts:
            pltpu.sync_copy(cbuf, cnt_sh.at[pl.ds(s * RPS, RPS)])
        plsc.subcore_barrier()

        def start_load(j, p):
            off = base + j * ch
            pltpu.async_copy(r_hbm.at[pl.ds(off, ch)], idx.at[p], sem_li[p])
            pltpu.async_copy(e_hbm.at[pl.ds(off, ch), pl.ds(lane0, DH)],
                             rows.at[p], sem_lr[p])

        def wait_load(p):
            pltpu.make_async_copy(r_hbm.at[pl.ds(0, ch)], idx.at[p],
                                  sem_li[p]).wait()
            pltpu.make_async_copy(e_hbm.at[pl.ds(0, ch), pl.ds(0, DH)],
                                  rows.at[p], sem_lr[p]).wait()

        def start_scat(p):
            pltpu.async_copy(rows.at[p], acc_sh.at[idx.at[p]], sem_sc[p],
                             add=True)
            if with_counts:
                @pl.when(c == cnt_core)
                def _():
                    pltpu.async_copy(ones, cnt_sh.at[idx.at[p]], sem_cn[p],
                                     add=True)

        def wait_scat(p):
            pltpu.make_async_copy(rows.at[p], acc_sh.at[idx.at[p]],
                                  sem_sc[p]).wait()
            if with_counts:
                @pl.when(c == cnt_core)
                def _():
                    pltpu.make_async_copy(ones, cnt_sh.at[idx.at[p]],
                                          sem_cn[p]).wait()

        start_load(0, 0)
        start_load(1, 1)

        def pair(jj, carry):
            for p in range(2):
                wait_load(p)
                start_scat(p)
            for p in range(2):
                j = jj * 2 + p
                wait_scat(p)

                @pl.when(j + 2 < nchs)
                def _():
                    start_load(j + 2, p)
            return carry
        lax.fori_loop(0, nchs // 2, pair, 0)
        plsc.subcore_barrier()

        for t in range(RPS // ZR):
            r0 = s * RPS + t * ZR
            pltpu.sync_copy(acc_sh.at[pl.ds(r0, ZR)], zbuf)
            pltpu.sync_copy(zbuf, part_hbm.at[pl.ds(r0, ZR),
                                              pl.ds(lane0, DH)])
        if with_counts:
            @pl.when(c == cnt_core)
            def _():
                pltpu.sync_copy(cnt_sh.at[pl.ds(s * RPS, RPS)], cbuf)
                pltpu.sync_copy(cbuf, cnt_hbm.at[pl.ds(s * RPS, RPS)])

    return pl.kernel(body, out_type=tuple(out_type), mesh=_MESH,
                     compiler_params=_SC_PARAMS, scratch_types=scratch)


EH = E // 2
_gather_half = _make_sc_gather(EH, 40)
_scatter_half_counts_a = _make_sc_scatter(True, EH, 40, cnt_core=0)
_scatter_half_counts_b = _make_sc_scatter(True, EH, 40, cnt_core=1)
_scatter_half = _make_sc_scatter(False, EH, 40)


def _mlp_step(x, W, b, g, beta):
    y = jnp.dot(x, W, preferred_element_type=jnp.float32) + b
    y = jax.nn.gelu(y)
    mu = jnp.mean(y, axis=-1, keepdims=True)
    var = jnp.mean((y - mu) ** 2, axis=-1, keepdims=True)
    return (y - mu) * lax.rsqrt(var + 1e-5) * g + beta


NB_N = 10
BN = N // NB_N
NB_E = 80
BE = E // NB_E


def _embed_body(x_ref, w_ref, b_ref, o_ref):
    o_ref[...] = (jnp.dot(x_ref[...], w_ref[...],
                          preferred_element_type=jnp.float32) + b_ref[...])


def _embed(x, Wn, bn):
    k = x.shape[1]
    return pl.pallas_call(
        _embed_body,
        grid=(NB_N,),
        in_specs=[
            pl.BlockSpec((BN, k), lambda i: (i, 0)),
            pl.BlockSpec((k, D), lambda i: (0, 0)),
            pl.BlockSpec((1, D), lambda i: (0, 0)),
        ],
        out_specs=pl.BlockSpec((BN, D), lambda i: (i, 0)),
        out_shape=jax.ShapeDtypeStruct((N, D), jnp.float32),
    )(x, Wn, bn)


def _make_edge(first):
    def body(e_ref, gg_ref, We_ref, be_ref, W_ref, b_ref, g_ref,
             beta_ref, o_ref):
        if first:
            ed = e_ref[...]
            msg = (ed[:, 0:1] * We_ref[0:1, :] + ed[:, 1:2] * We_ref[1:2, :]
                   + be_ref[...])
        else:
            msg = e_ref[...]
        msg = msg + gg_ref[...].astype(jnp.float32)
        for i in range(2):
            msg = _mlp_step(msg, W_ref[i], b_ref[i], g_ref[i], beta_ref[i])
        o_ref[...] = msg

    ein = 2 if first else D

    def call(e, gg, We, be, W, b, g, beta):
        ne = gg.shape[0]
        return pl.pallas_call(
            body,
            grid=(ne // BE,),
            in_specs=[
                pl.BlockSpec((BE, ein), lambda i: (i, 0)),
                pl.BlockSpec((BE, D), lambda i: (i, 0)),
                pl.BlockSpec((2, D), lambda i: (0, 0)),
                pl.BlockSpec((1, D), lambda i: (0, 0)),
                pl.BlockSpec((2, D, D), lambda i: (0, 0, 0)),
                pl.BlockSpec((2, D), lambda i: (0, 0)),
                pl.BlockSpec((2, D), lambda i: (0, 0)),
                pl.BlockSpec((2, D), lambda i: (0, 0)),
            ],
            out_specs=pl.BlockSpec((BE, D), lambda i: (i, 0)),
            out_shape=jax.ShapeDtypeStruct((ne, D), jnp.float32),
        )(e, gg, We, be, W, b, g, beta)

    return call


_edge_first = _make_edge(True)
_edge_rest = _make_edge(False)


def _make_node(last):
    def body(h_ref, pa_ref, pb_ref, ca_ref, cb_ref, W_ref, b_ref, g_ref,
             beta_ref, *rest):
        if last:
            (oW_ref, ob_ref, og_ref, obeta_ref, pjW_ref, pjb_ref,
             o_ref) = rest
        else:
            (o_ref,) = rest
        cnt = ca_ref[:, 0:1] + cb_ref[:, 0:1]
        x = h_ref[...] + (pa_ref[...] + pb_ref[...]) / jnp.maximum(cnt, 1.0)
        for i in range(2):
            x = _mlp_step(x, W_ref[i], b_ref[i], g_ref[i], beta_ref[i])
        if last:
            x = _mlp_step(x, oW_ref[...], ob_ref[...], og_ref[...],
                          obeta_ref[...])
            x = (jnp.dot(x, pjW_ref[...], preferred_element_type=jnp.float32)
                 + pjb_ref[...])
        o_ref[...] = x

    PD = PRED * COUT

    def call(h, pa, pb, ca, cb, W, b, g, beta, *rest):
        in_specs = [
            pl.BlockSpec((BN, D), lambda i: (i, 0)),
            pl.BlockSpec((BN, D), lambda i: (i, 0)),
            pl.BlockSpec((BN, D), lambda i: (i, 0)),
            pl.BlockSpec((BN, CL), lambda i: (i, 0)),
            pl.BlockSpec((BN, CL), lambda i: (i, 0)),
            pl.BlockSpec((2, D, D), lambda i: (0, 0, 0)),
            pl.BlockSpec((2, D), lambda i: (0, 0)),
            pl.BlockSpec((2, D), lambda i: (0, 0)),
            pl.BlockSpec((2, D), lambda i: (0, 0)),
        ]
        if last:
            in_specs += [
                pl.BlockSpec((D, D), lambda i: (0, 0)),
                pl.BlockSpec((1, D), lambda i: (0, 0)),
                pl.BlockSpec((1, D), lambda i: (0, 0)),
                pl.BlockSpec((1, D), lambda i: (0, 0)),
                pl.BlockSpec((D, PD), lambda i: (0, 0)),
                pl.BlockSpec((1, PD), lambda i: (0, 0)),
            ]
            out_w = PD
        else:
            out_w = D
        return pl.pallas_call(
            body,
            grid=(NB_N,),
            in_specs=in_specs,
            out_specs=pl.BlockSpec((BN, out_w), lambda i: (i, 0)),
            out_shape=jax.ShapeDtypeStruct((N, out_w), jnp.float32),
        )(h, pa, pb, ca, cb, W, b, g, beta, *rest)

    return call


_node_mid = _make_node(False)
_node_last = _make_node(True)


def kernel(nodes, edges, senders, receivers, Wn, bn, We, be,
           eW, eb, eg, ebeta, nW, nb, ng, nbeta,
           oW, ob, og, obeta, pW, pb):
    x = nodes.reshape(N, -1)
    senders = senders.astype(jnp.int32)
    receivers = receivers.astype(jnp.int32)
    h = _embed(x, Wn, bn.reshape(1, D))

    sA, sB = senders[:EH], senders[EH:]
    rA, rB = receivers[:EH], receivers[EH:]

    gA = _gather_half(h, sA, rA)
    gB = _gather_half(h, sB, rB)
    eA = _edge_first(edges[:EH], gA, We, be.reshape(1, D),
                     eW[0], eb[0], eg[0], ebeta[0])
    eB = _edge_first(edges[EH:], gB, We, be.reshape(1, D),
                     eW[0], eb[0], eg[0], ebeta[0])
    pA, cA = _scatter_half_counts_a(eA, rA)
    pB, cB = _scatter_half_counts_b(eB, rB)
    h = _node_mid(h, pA, pB, cA, cB, nW[0], nb[0], ng[0], nbeta[0])

    gA = _gather_half(h, sA, rA)
    gB = _gather_half(h, sB, rB)
    eA = _edge_rest(eA, gA, We, be.reshape(1, D),
                    eW[1], eb[1], eg[1], ebeta[1])
    eB = _edge_rest(eB, gB, We, be.reshape(1, D),
                    eW[1], eb[1], eg[1], ebeta[1])
    (pA,) = _scatter_half(eA, rA)
    (pB,) = _scatter_half(eB, rB)
    out = _node_last(h, pA, pB, cA, cB, nW[1], nb[1], ng[1], nbeta[1],
                     oW, ob.reshape(1, D), og.reshape(1, D),
                     obeta.reshape(1, D), pW, pb.reshape(1, PRED * COUT))
    return out.reshape(N, PRED, COUT)


# speedup vs baseline: 1.0302x; 1.0302x over previous
import functools

import jax
import jax.numpy as jnp
from jax import lax
from jax.experimental import pallas as pl
from jax.experimental.pallas import tpu as pltpu
from jax.experimental.pallas import tpu_sc as plsc

N = 10000
E = 320000
D = 128
PRED = 24
COUT = 7

NC = 2
NS = 16
NW = NC * NS
EPW = E // NW
CH = 80
NCHUNK = EPW // CH
DH = D // NC
EPS = E // NS
NCH_S = EPS // CH
NPAD = 10240
RPS = NPAD // NS
ZR = 128
CL = 16

_MESH = plsc.VectorSubcoreMesh(
    core_axis_name="c", subcore_axis_name="s", num_cores=NC, num_subcores=NS)
_SC_PARAMS = pltpu.CompilerParams(use_tc_tiling_on_sc=False)


def _make_sc_gather(ne, ch):
    epw = ne // NW
    nch = epw // ch
    assert epw % ch == 0 and ch % 8 == 0 and ch <= 128

    def body(h_hbm, s_hbm, r_hbm, g_hbm, idx_s, idx_r, rows_s, rows_r,
             h_sh, *sems):
        sem_gs, sem_gr = sems[0:2], sems[2:4]
        sem_ws = sems[4:6]
        c = lax.axis_index("c")
        s = lax.axis_index("s")
        wid = s * NC + c
        base = wid * epw

        pltpu.sync_copy(s_hbm.at[pl.ds(wid * nch, nch)], idx_s)
        pltpu.sync_copy(r_hbm.at[pl.ds(wid * nch, nch)], idx_r)

        nps = N // NS
        pltpu.sync_copy(h_hbm.at[pl.ds(s * nps, nps)],
                        h_sh.at[pl.ds(s * nps, nps)])
        plsc.subcore_barrier()

        def start_gather(j, p):
            pltpu.async_copy(h_sh.at[idx_s.at[j]], rows_s.at[p], sem_gs[p])
            pltpu.async_copy(h_sh.at[idx_r.at[j]], rows_r.at[p], sem_gr[p])

        def wait_gather(j, p):
            pltpu.make_async_copy(h_sh.at[idx_s.at[j]], rows_s.at[p],
                                  sem_gs[p]).wait()
            pltpu.make_async_copy(h_sh.at[idx_r.at[j]], rows_r.at[p],
                                  sem_gr[p]).wait()

        def addrows(p):
            @plsc.parallel_loop(0, ch, unroll=4)
            def _(i):
                for k in range(D // 32):
                    sl = pl.ds(k * 32, 32)
                    rows_s[p, i, sl] = rows_s[p, i, sl] + rows_r[p, i, sl]

        def start_wb(j, p):
            off = base + j * ch
            pltpu.async_copy(rows_s.at[p], g_hbm.at[pl.ds(off, ch)],
                             sem_ws[p])

        def wait_wb(p):
            pltpu.make_async_copy(rows_s.at[p], g_hbm.at[pl.ds(0, ch)],
                                  sem_ws[p]).wait()

        def pair(jj, carry):
            j0 = jj * 2
            for p in range(2):
                start_gather(j0 + p, p)
            for p in range(2):
                j = j0 + p
                wait_gather(j, p)
                addrows(p)
                start_wb(j, p)
            for p in range(2):
                wait_wb(p)
            return carry

        lax.fori_loop(0, nch // 2, pair, 0)
        if nch % 2:
            start_gather(nch - 1, 0)
            wait_gather(nch - 1, 0)
            addrows(0)
            start_wb(nch - 1, 0)
            wait_wb(0)

    k = pl.kernel(
        body,
        out_type=jax.ShapeDtypeStruct((ne, D), jnp.bfloat16),
        mesh=_MESH,
        compiler_params=_SC_PARAMS,
        scratch_types=[
            pltpu.VMEM((nch, ch), jnp.int32),
            pltpu.VMEM((nch, ch), jnp.int32),
            pltpu.VMEM((2, ch, D), jnp.bfloat16),
            pltpu.VMEM((2, ch, D), jnp.bfloat16),
            pltpu.VMEM_SHARED((N, D), jnp.bfloat16),
        ] + [pltpu.SemaphoreType.DMA] * 6,
    )

    def call(h, s, r):
        return k(h.astype(jnp.bfloat16), s.reshape(-1, ch),
                 r.reshape(-1, ch))

    return call


def _make_sc_scatter(with_counts, ne, ch, cnt_core=0):
    eps = ne // NS
    nchs = eps // ch
    assert eps % ch == 0 and nchs % 2 == 0 and ch % 8 == 0 and ch <= 128
    out_type = [jax.ShapeDtypeStruct((NPAD, D), jnp.float32)]
    scratch = [
        pltpu.VMEM((2, ch), jnp.int32),
        pltpu.VMEM((2, ch, DH), jnp.float32),
        pltpu.VMEM((ZR, DH), jnp.float32),
        pltpu.VMEM_SHARED((NPAD, DH), jnp.float32),
    ] + [pltpu.SemaphoreType.DMA] * 8
    if with_counts:
        out_type.append(jax.ShapeDtypeStruct((NPAD, CL), jnp.float32))
        scratch += [
            pltpu.VMEM((ch, CL), jnp.float32),
            pltpu.VMEM((RPS, CL), jnp.float32),
            pltpu.VMEM_SHARED((NPAD, CL), jnp.float32),
        ]

    def body(e_hbm, r_hbm, *refs):
        if with_counts:
            (part_hbm, cnt_hbm, idx, rows, zbuf, acc_sh,
             sli0, sli1, slr0, slr1, ssc0, ssc1, scn0, scn1,
             ones, cbuf, cnt_sh) = refs
        else:
            (part_hbm, idx, rows, zbuf, acc_sh,
             sli0, sli1, slr0, slr1, ssc0, ssc1, scn0, scn1) = refs
        sem_li, sem_lr = (sli0, sli1), (slr0, slr1)
        sem_sc, sem_cn = (ssc0, ssc1), (scn0, scn1)
        c = lax.axis_index("c")
        s = lax.axis_index("s")
        base = s * eps
        lane0 = c * DH
        zero16 = jnp.zeros((16,), jnp.float32)
        one16 = jnp.ones((16,), jnp.float32)

        def zrow(i, carry):
            for k in range(DH // 16):
                zbuf[i, pl.ds(k * 16, 16)] = zero16
            return carry
        lax.fori_loop(0, ZR, zrow, 0)
        if with_counts:
            def crow(i, carry):
                cbuf[i, pl.ds(0, 16)] = zero16
                return carry
            lax.fori_loop(0, RPS, crow, 0)

            def orow(i, carry):
                ones[i, pl.ds(0, 16)] = one16
                return carry
            lax.fori_loop(0, ch, orow, 0)

        for t in range(RPS // ZR):
            pltpu.sync_copy(zbuf, acc_sh.at[pl.ds(s * RPS + t * ZR, ZR)])
        if with_counts:
            pltpu.sync_copy(cbuf, cnt_sh.at[pl.ds(s * RPS, RPS)])
        plsc.subcore_barrier()

        def start_load(j, p):
            off = base + j * ch
            pltpu.async_copy(r_hbm.at[pl.ds(off, ch)], idx.at[p], sem_li[p])
            pltpu.async_copy(e_hbm.at[pl.ds(off, ch), pl.ds(lane0, DH)],
                             rows.at[p], sem_lr[p])

        def wait_load(p):
            pltpu.make_async_copy(r_hbm.at[pl.ds(0, ch)], idx.at[p],
                                  sem_li[p]).wait()
            pltpu.make_async_copy(e_hbm.at[pl.ds(0, ch), pl.ds(0, DH)],
                                  rows.at[p], sem_lr[p]).wait()

        def start_scat(p):
            pltpu.async_copy(rows.at[p], acc_sh.at[idx.at[p]], sem_sc[p],
                             add=True)
            if with_counts:
                @pl.when(c == cnt_core)
                def _():
                    pltpu.async_copy(ones, cnt_sh.at[idx.at[p]], sem_cn[p],
                                     add=True)

        def wait_scat(p):
            pltpu.make_async_copy(rows.at[p], acc_sh.at[idx.at[p]],
                                  sem_sc[p]).wait()
            if with_counts:
                @pl.when(c == cnt_core)
                def _():
                    pltpu.make_async_copy(ones, cnt_sh.at[idx.at[p]],
                                          sem_cn[p]).wait()

        start_load(0, 0)
        start_load(1, 1)

        def pair(jj, carry):
            for p in range(2):
                wait_load(p)
                start_scat(p)
            for p in range(2):
                j = jj * 2 + p
                wait_scat(p)

                @pl.when(j + 2 < nchs)
                def _():
                    start_load(j + 2, p)
            return carry
        lax.fori_loop(0, nchs // 2, pair, 0)
        plsc.subcore_barrier()

        for t in range(RPS // ZR):
            r0 = s * RPS + t * ZR
            pltpu.sync_copy(acc_sh.at[pl.ds(r0, ZR)], zbuf)
            pltpu.sync_copy(zbuf, part_hbm.at[pl.ds(r0, ZR),
                                              pl.ds(lane0, DH)])
        if with_counts:
            @pl.when(c == cnt_core)
            def _():
                pltpu.sync_copy(cnt_sh.at[pl.ds(s * RPS, RPS)], cbuf)
                pltpu.sync_copy(cbuf, cnt_hbm.at[pl.ds(s * RPS, RPS)])

    return pl.kernel(body, out_type=tuple(out_type), mesh=_MESH,
                     compiler_params=_SC_PARAMS, scratch_types=scratch)


EH = E // 2
_gather_half = _make_sc_gather(EH, 40)
_scatter_half_counts_a = _make_sc_scatter(True, EH, 40, cnt_core=0)
_scatter_half_counts_b = _make_sc_scatter(True, EH, 40, cnt_core=1)
_scatter_half = _make_sc_scatter(False, EH, 40)


def _mlp_step(x, W, b, g, beta):
    y = jnp.dot(x, W, preferred_element_type=jnp.float32) + b
    y = jax.nn.gelu(y)
    mu = jnp.mean(y, axis=-1, keepdims=True)
    var = jnp.mean((y - mu) ** 2, axis=-1, keepdims=True)
    return (y - mu) * lax.rsqrt(var + 1e-5) * g + beta


NB_N = 10
BN = N // NB_N
NB_E = 80
BE = E // NB_E


def _embed_body(x_ref, w_ref, b_ref, o_ref):
    o_ref[...] = (jnp.dot(x_ref[...], w_ref[...],
                          preferred_element_type=jnp.float32) + b_ref[...])


def _embed(x, Wn, bn):
    k = x.shape[1]
    return pl.pallas_call(
        _embed_body,
        grid=(NB_N,),
        in_specs=[
            pl.BlockSpec((BN, k), lambda i: (i, 0)),
            pl.BlockSpec((k, D), lambda i: (0, 0)),
            pl.BlockSpec((1, D), lambda i: (0, 0)),
        ],
        out_specs=pl.BlockSpec((BN, D), lambda i: (i, 0)),
        out_shape=jax.ShapeDtypeStruct((N, D), jnp.float32),
    )(x, Wn, bn)


def _make_edge(first):
    def body(e_ref, gg_ref, We_ref, be_ref, W_ref, b_ref, g_ref,
             beta_ref, o_ref):
        if first:
            ed = e_ref[...]
            msg = (ed[:, 0:1] * We_ref[0:1, :] + ed[:, 1:2] * We_ref[1:2, :]
                   + be_ref[...])
        else:
            msg = e_ref[...]
        msg = msg + gg_ref[...].astype(jnp.float32)
        for i in range(2):
            msg = _mlp_step(msg, W_ref[i], b_ref[i], g_ref[i], beta_ref[i])
        o_ref[...] = msg

    ein = 2 if first else D

    def call(e, gg, We, be, W, b, g, beta):
        ne = gg.shape[0]
        return pl.pallas_call(
            body,
            grid=(ne // BE,),
            in_specs=[
                pl.BlockSpec((BE, ein), lambda i: (i, 0)),
                pl.BlockSpec((BE, D), lambda i: (i, 0)),
                pl.BlockSpec((2, D), lambda i: (0, 0)),
                pl.BlockSpec((1, D), lambda i: (0, 0)),
                pl.BlockSpec((2, D, D), lambda i: (0, 0, 0)),
                pl.BlockSpec((2, D), lambda i: (0, 0)),
                pl.BlockSpec((2, D), lambda i: (0, 0)),
                pl.BlockSpec((2, D), lambda i: (0, 0)),
            ],
            out_specs=pl.BlockSpec((BE, D), lambda i: (i, 0)),
            out_shape=jax.ShapeDtypeStruct((ne, D), jnp.float32),
        )(e, gg, We, be, W, b, g, beta)

    return call


_edge_first = _make_edge(True)
_edge_rest = _make_edge(False)


def _make_node(last):
    def body(h_ref, pa_ref, pb_ref, ca_ref, cb_ref, W_ref, b_ref, g_ref,
             beta_ref, *rest):
        if last:
            (oW_ref, ob_ref, og_ref, obeta_ref, pjW_ref, pjb_ref,
             o_ref) = rest
        else:
            (o_ref,) = rest
        cnt = ca_ref[:, 0:1] + cb_ref[:, 0:1]
        x = h_ref[...] + (pa_ref[...] + pb_ref[...]) / jnp.maximum(cnt, 1.0)
        for i in range(2):
            x = _mlp_step(x, W_ref[i], b_ref[i], g_ref[i], beta_ref[i])
        if last:
            x = _mlp_step(x, oW_ref[...], ob_ref[...], og_ref[...],
                          obeta_ref[...])
            x = (jnp.dot(x, pjW_ref[...], preferred_element_type=jnp.float32)
                 + pjb_ref[...])
        o_ref[...] = x

    PD = PRED * COUT

    def call(h, pa, pb, ca, cb, W, b, g, beta, *rest):
        in_specs = [
            pl.BlockSpec((BN, D), lambda i: (i, 0)),
            pl.BlockSpec((BN, D), lambda i: (i, 0)),
            pl.BlockSpec((BN, D), lambda i: (i, 0)),
            pl.BlockSpec((BN, CL), lambda i: (i, 0)),
            pl.BlockSpec((BN, CL), lambda i: (i, 0)),
            pl.BlockSpec((2, D, D), lambda i: (0, 0, 0)),
            pl.BlockSpec((2, D), lambda i: (0, 0)),
            pl.BlockSpec((2, D), lambda i: (0, 0)),
            pl.BlockSpec((2, D), lambda i: (0, 0)),
        ]
        if last:
            in_specs += [
                pl.BlockSpec((D, D), lambda i: (0, 0)),
                pl.BlockSpec((1, D), lambda i: (0, 0)),
                pl.BlockSpec((1, D), lambda i: (0, 0)),
                pl.BlockSpec((1, D), lambda i: (0, 0)),
                pl.BlockSpec((D, PD), lambda i: (0, 0)),
                pl.BlockSpec((1, PD), lambda i: (0, 0)),
            ]
            out_w = PD
        else:
            out_w = D
        return pl.pallas_call(
            body,
            grid=(NB_N,),
            in_specs=in_specs,
            out_specs=pl.BlockSpec((BN, out_w), lambda i: (i, 0)),
            out_shape=jax.ShapeDtypeStruct((N, out_w), jnp.float32),
        )(h, pa, pb, ca, cb, W, b, g, beta, *rest)

    return call


_node_mid = _make_node(False)
_node_last = _make_node(True)


def kernel(nodes, edges, senders, receivers, Wn, bn, We, be,
           eW, eb, eg, ebeta, nW, nb, ng, nbeta,
           oW, ob, og, obeta, pW, pb):
    x = nodes.reshape(N, -1)
    senders = senders.astype(jnp.int32)
    receivers = receivers.astype(jnp.int32)
    h = _embed(x, Wn, bn.reshape(1, D))

    sA, sB = senders[:EH], senders[EH:]
    rA, rB = receivers[:EH], receivers[EH:]

    gA = _gather_half(h, sA, rA)
    gB = _gather_half(h, sB, rB)
    eA = _edge_first(edges[:EH], gA, We, be.reshape(1, D),
                     eW[0], eb[0], eg[0], ebeta[0])
    eB = _edge_first(edges[EH:], gB, We, be.reshape(1, D),
                     eW[0], eb[0], eg[0], ebeta[0])
    pA, cA = _scatter_half_counts_a(eA, rA)
    pB, cB = _scatter_half_counts_b(eB, rB)
    h = _node_mid(h, pA, pB, cA, cB, nW[0], nb[0], ng[0], nbeta[0])

    gA = _gather_half(h, sA, rA)
    gB = _gather_half(h, sB, rB)
    eA = _edge_rest(eA, gA, We, be.reshape(1, D),
                    eW[1], eb[1], eg[1], ebeta[1])
    eB = _edge_rest(eB, gB, We, be.reshape(1, D),
                    eW[1], eb[1], eg[1], ebeta[1])
    (pA,) = _scatter_half(eA, rA)
    (pB,) = _scatter_half(eB, rB)
    out = _node_last(h, pA, pB, cA, cB, nW[1], nb[1], ng[1], nbeta[1],
                     oW, ob.reshape(1, D), og.reshape(1, D),
                     obeta.reshape(1, D), pW, pb.reshape(1, PRED * COUT))
    return out.reshape(N, PRED, COUT)


# speedup vs baseline: 1.5176x; 1.4732x over previous
import functools

import jax
import jax.numpy as jnp
from jax import lax
from jax.experimental import pallas as pl
from jax.experimental.pallas import tpu as pltpu
from jax.experimental.pallas import tpu_sc as plsc

N = 10000
E = 320000
D = 128
PRED = 24
COUT = 7

NC = 2
NS = 16
NW = NC * NS
EPW = E // NW
CH = 80
NCHUNK = EPW // CH
DH = D // NC
EPS = E // NS
NCH_S = EPS // CH
NPAD = 10240
RPS = NPAD // NS
ZR = 128
CL = 16

_MESH = plsc.VectorSubcoreMesh(
    core_axis_name="c", subcore_axis_name="s", num_cores=NC, num_subcores=NS)
_SC_PARAMS = pltpu.CompilerParams(use_tc_tiling_on_sc=False)


def _make_sc_gather(ne, ch):
    epw = ne // NW
    nch = epw // ch
    assert epw % ch == 0 and ch % 8 == 0 and ch <= 128

    def body(h_hbm, s_hbm, r_hbm, g_hbm, idx_s, idx_r, rows_s, rows_r,
             h_sh, *sems):
        sem_gs, sem_gr = sems[0:2], sems[2:4]
        sem_ws = sems[4:6]
        c = lax.axis_index("c")
        s = lax.axis_index("s")
        wid = s * NC + c
        base = wid * epw

        pltpu.sync_copy(s_hbm.at[pl.ds(wid * nch, nch)], idx_s)
        pltpu.sync_copy(r_hbm.at[pl.ds(wid * nch, nch)], idx_r)

        nps = N // NS
        pltpu.sync_copy(h_hbm.at[pl.ds(s * nps, nps)],
                        h_sh.at[pl.ds(s * nps, nps)])
        plsc.subcore_barrier()

        def start_gather(j, p):
            pltpu.async_copy(h_sh.at[idx_s.at[j]], rows_s.at[p], sem_gs[p])
            pltpu.async_copy(h_sh.at[idx_r.at[j]], rows_r.at[p], sem_gr[p])

        def wait_gather(j, p):
            pltpu.make_async_copy(h_sh.at[idx_s.at[j]], rows_s.at[p],
                                  sem_gs[p]).wait()
            pltpu.make_async_copy(h_sh.at[idx_r.at[j]], rows_r.at[p],
                                  sem_gr[p]).wait()

        def addrows(p):
            @plsc.parallel_loop(0, ch, unroll=4)
            def _(i):
                for k in range(D // 16):
                    sl = pl.ds(k * 16, 16)
                    rows_s[p, i, sl] = rows_s[p, i, sl] + rows_r[p, i, sl]

        def start_wb(j, p):
            off = base + j * ch
            pltpu.async_copy(rows_s.at[p], g_hbm.at[pl.ds(off, ch)],
                             sem_ws[p])

        def wait_wb(p):
            pltpu.make_async_copy(rows_s.at[p], g_hbm.at[pl.ds(0, ch)],
                                  sem_ws[p]).wait()

        def pair(jj, carry):
            j0 = jj * 2
            for p in range(2):
                start_gather(j0 + p, p)
            for p in range(2):
                j = j0 + p
                wait_gather(j, p)
                addrows(p)
                start_wb(j, p)
            for p in range(2):
                wait_wb(p)
            return carry

        lax.fori_loop(0, nch // 2, pair, 0)
        if nch % 2:
            start_gather(nch - 1, 0)
            wait_gather(nch - 1, 0)
            addrows(0)
            start_wb(nch - 1, 0)
            wait_wb(0)

    k = pl.kernel(
        body,
        out_type=jax.ShapeDtypeStruct((ne, D), jnp.float32),
        mesh=_MESH,
        compiler_params=_SC_PARAMS,
        scratch_types=[
            pltpu.VMEM((nch, ch), jnp.int32),
            pltpu.VMEM((nch, ch), jnp.int32),
            pltpu.VMEM((2, ch, D), jnp.float32),
            pltpu.VMEM((2, ch, D), jnp.float32),
            pltpu.VMEM_SHARED((N, D), jnp.float32),
        ] + [pltpu.SemaphoreType.DMA] * 6,
    )

    def call(h, s, r):
        return k(h, s.reshape(-1, ch), r.reshape(-1, ch))

    return call


def _make_sc_scatter(with_counts, ne, ch, cnt_core=0):
    eps = ne // NS
    nchs = eps // ch
    assert eps % ch == 0 and nchs % 2 == 0 and ch % 8 == 0 and ch <= 128
    out_type = [jax.ShapeDtypeStruct((NPAD, D), jnp.float32)]
    scratch = [
        pltpu.VMEM((2, ch), jnp.int32),
        pltpu.VMEM((2, ch, DH), jnp.float32),
        pltpu.VMEM((ZR, DH), jnp.float32),
        pltpu.VMEM_SHARED((NPAD, DH), jnp.float32),
    ] + [pltpu.SemaphoreType.DMA] * 8
    if with_counts:
        out_type.append(jax.ShapeDtypeStruct((NPAD, CL), jnp.float32))
        scratch += [
            pltpu.VMEM((ch, CL), jnp.float32),
            pltpu.VMEM((RPS, CL), jnp.float32),
            pltpu.VMEM_SHARED((NPAD, CL), jnp.float32),
        ]

    def body(e_hbm, r_hbm, *refs):
        if with_counts:
            (part_hbm, cnt_hbm, idx, rows, zbuf, acc_sh,
             sli0, sli1, slr0, slr1, ssc0, ssc1, scn0, scn1,
             ones, cbuf, cnt_sh) = refs
        else:
            (part_hbm, idx, rows, zbuf, acc_sh,
             sli0, sli1, slr0, slr1, ssc0, ssc1, scn0, scn1) = refs
        sem_li, sem_lr = (sli0, sli1), (slr0, slr1)
        sem_sc, sem_cn = (ssc0, ssc1), (scn0, scn1)
        c = lax.axis_index("c")
        s = lax.axis_index("s")
        base = s * eps
        lane0 = c * DH
        zero16 = jnp.zeros((16,), jnp.float32)
        one16 = jnp.ones((16,), jnp.float32)

        def zrow(i, carry):
            for k in range(DH // 16):
                zbuf[i, pl.ds(k * 16, 16)] = zero16
            return carry
        lax.fori_loop(0, ZR, zrow, 0)
        if with_counts:
            def crow(i, carry):
                cbuf[i, pl.ds(0, 16)] = zero16
                return carry
            lax.fori_loop(0, RPS, crow, 0)

            def orow(i, carry):
                ones[i, pl.ds(0, 16)] = one16
                return carry
            lax.fori_loop(0, ch, orow, 0)

        for t in range(RPS // ZR):
            pltpu.sync_copy(zbuf, acc_sh.at[pl.ds(s * RPS + t * ZR, ZR)])
        if with_counts:
            pltpu.sync_copy(cbuf, cnt_sh.at[pl.ds(s * RPS, RPS)])
        plsc.subcore_barrier()

        def start_load(j, p):
            off = base + j * ch
            pltpu.async_copy(r_hbm.at[pl.ds(off, ch)], idx.at[p], sem_li[p])
            pltpu.async_copy(e_hbm.at[pl.ds(off, ch), pl.ds(lane0, DH)],
                             rows.at[p], sem_lr[p])

        def wait_load(p):
            pltpu.make_async_copy(r_hbm.at[pl.ds(0, ch)], idx.at[p],
                                  sem_li[p]).wait()
            pltpu.make_async_copy(e_hbm.at[pl.ds(0, ch), pl.ds(0, DH)],
                                  rows.at[p], sem_lr[p]).wait()

        def start_scat(p):
            pltpu.async_copy(rows.at[p], acc_sh.at[idx.at[p]], sem_sc[p],
                             add=True)
            if with_counts:
                @pl.when(c == cnt_core)
                def _():
                    pltpu.async_copy(ones, cnt_sh.at[idx.at[p]], sem_cn[p],
                                     add=True)

        def wait_scat(p):
            pltpu.make_async_copy(rows.at[p], acc_sh.at[idx.at[p]],
                                  sem_sc[p]).wait()
            if with_counts:
                @pl.when(c == cnt_core)
                def _():
                    pltpu.make_async_copy(ones, cnt_sh.at[idx.at[p]],
                                          sem_cn[p]).wait()

        start_load(0, 0)
        start_load(1, 1)

        def pair(jj, carry):
            for p in range(2):
                wait_load(p)
                start_scat(p)
            for p in range(2):
                j = jj * 2 + p
                wait_scat(p)

                @pl.when(j + 2 < nchs)
                def _():
                    start_load(j + 2, p)
            return carry
        lax.fori_loop(0, nchs // 2, pair, 0)
        plsc.subcore_barrier()

        for t in range(RPS // ZR):
            r0 = s * RPS + t * ZR
            pltpu.sync_copy(acc_sh.at[pl.ds(r0, ZR)], zbuf)
            pltpu.sync_copy(zbuf, part_hbm.at[pl.ds(r0, ZR),
                                              pl.ds(lane0, DH)])
        if with_counts:
            @pl.when(c == cnt_core)
            def _():
                pltpu.sync_copy(cnt_sh.at[pl.ds(s * RPS, RPS)], cbuf)
                pltpu.sync_copy(cbuf, cnt_hbm.at[pl.ds(s * RPS, RPS)])

    return pl.kernel(body, out_type=tuple(out_type), mesh=_MESH,
                     compiler_params=_SC_PARAMS, scratch_types=scratch)


EH = E // 2
_gather_half = _make_sc_gather(EH, 40)
_scatter_half_counts_a = _make_sc_scatter(True, EH, 40, cnt_core=0)
_scatter_half_counts_b = _make_sc_scatter(True, EH, 40, cnt_core=1)
_scatter_half = _make_sc_scatter(False, EH, 40)


def _mlp_step(x, W, b, g, beta):
    y = jnp.dot(x, W, preferred_element_type=jnp.float32) + b
    y = jax.nn.gelu(y)
    mu = jnp.mean(y, axis=-1, keepdims=True)
    var = jnp.mean((y - mu) ** 2, axis=-1, keepdims=True)
    return (y - mu) * lax.rsqrt(var + 1e-5) * g + beta


NB_N = 10
BN = N // NB_N
NB_E = 80
BE = E // NB_E


def _embed_body(x_ref, w_ref, b_ref, o_ref):
    o_ref[...] = (jnp.dot(x_ref[...], w_ref[...],
                          preferred_element_type=jnp.float32) + b_ref[...])


def _embed(x, Wn, bn):
    k = x.shape[1]
    return pl.pallas_call(
        _embed_body,
        grid=(NB_N,),
        in_specs=[
            pl.BlockSpec((BN, k), lambda i: (i, 0)),
            pl.BlockSpec((k, D), lambda i: (0, 0)),
            pl.BlockSpec((1, D), lambda i: (0, 0)),
        ],
        out_specs=pl.BlockSpec((BN, D), lambda i: (i, 0)),
        out_shape=jax.ShapeDtypeStruct((N, D), jnp.float32),
    )(x, Wn, bn)


def _make_edge(first):
    def body(e_ref, gg_ref, We_ref, be_ref, W_ref, b_ref, g_ref,
             beta_ref, o_ref):
        if first:
            ed = e_ref[...]
            msg = (ed[:, 0:1] * We_ref[0:1, :] + ed[:, 1:2] * We_ref[1:2, :]
                   + be_ref[...])
        else:
            msg = e_ref[...]
        msg = msg + gg_ref[...]
        for i in range(2):
            msg = _mlp_step(msg, W_ref[i], b_ref[i], g_ref[i], beta_ref[i])
        o_ref[...] = msg

    ein = 2 if first else D

    def call(e, gg, We, be, W, b, g, beta):
        ne = gg.shape[0]
        return pl.pallas_call(
            body,
            grid=(ne // BE,),
            in_specs=[
                pl.BlockSpec((BE, ein), lambda i: (i, 0)),
                pl.BlockSpec((BE, D), lambda i: (i, 0)),
                pl.BlockSpec((2, D), lambda i: (0, 0)),
                pl.BlockSpec((1, D), lambda i: (0, 0)),
                pl.BlockSpec((2, D, D), lambda i: (0, 0, 0)),
                pl.BlockSpec((2, D), lambda i: (0, 0)),
                pl.BlockSpec((2, D), lambda i: (0, 0)),
                pl.BlockSpec((2, D), lambda i: (0, 0)),
            ],
            out_specs=pl.BlockSpec((BE, D), lambda i: (i, 0)),
            out_shape=jax.ShapeDtypeStruct((ne, D), jnp.float32),
        )(e, gg, We, be, W, b, g, beta)

    return call


_edge_first = _make_edge(True)
_edge_rest = _make_edge(False)


def _make_node(last):
    def body(h_ref, pa_ref, pb_ref, ca_ref, cb_ref, W_ref, b_ref, g_ref,
             beta_ref, *rest):
        if last:
            (oW_ref, ob_ref, og_ref, obeta_ref, pjW_ref, pjb_ref,
             o_ref) = rest
        else:
            (o_ref,) = rest
        cnt = ca_ref[:, 0:1] + cb_ref[:, 0:1]
        x = h_ref[...] + (pa_ref[...] + pb_ref[...]) / jnp.maximum(cnt, 1.0)
        for i in range(2):
            x = _mlp_step(x, W_ref[i], b_ref[i], g_ref[i], beta_ref[i])
        if last:
            x = _mlp_step(x, oW_ref[...], ob_ref[...], og_ref[...],
                          obeta_ref[...])
            x = (jnp.dot(x, pjW_ref[...], preferred_element_type=jnp.float32)
                 + pjb_ref[...])
        o_ref[...] = x

    PD = PRED * COUT

    def call(h, pa, pb, ca, cb, W, b, g, beta, *rest):
        in_specs = [
            pl.BlockSpec((BN, D), lambda i: (i, 0)),
            pl.BlockSpec((BN, D), lambda i: (i, 0)),
            pl.BlockSpec((BN, D), lambda i: (i, 0)),
            pl.BlockSpec((BN, CL), lambda i: (i, 0)),
            pl.BlockSpec((BN, CL), lambda i: (i, 0)),
            pl.BlockSpec((2, D, D), lambda i: (0, 0, 0)),
            pl.BlockSpec((2, D), lambda i: (0, 0)),
            pl.BlockSpec((2, D), lambda i: (0, 0)),
            pl.BlockSpec((2, D), lambda i: (0, 0)),
        ]
        if last:
            in_specs += [
                pl.BlockSpec((D, D), lambda i: (0, 0)),
                pl.BlockSpec((1, D), lambda i: (0, 0)),
                pl.BlockSpec((1, D), lambda i: (0, 0)),
                pl.BlockSpec((1, D), lambda i: (0, 0)),
                pl.BlockSpec((D, PD), lambda i: (0, 0)),
                pl.BlockSpec((1, PD), lambda i: (0, 0)),
            ]
            out_w = PD
        else:
            out_w = D
        return pl.pallas_call(
            body,
            grid=(NB_N,),
            in_specs=in_specs,
            out_specs=pl.BlockSpec((BN, out_w), lambda i: (i, 0)),
            out_shape=jax.ShapeDtypeStruct((N, out_w), jnp.float32),
        )(h, pa, pb, ca, cb, W, b, g, beta, *rest)

    return call


_node_mid = _make_node(False)
_node_last = _make_node(True)


def kernel(nodes, edges, senders, receivers, Wn, bn, We, be,
           eW, eb, eg, ebeta, nW, nb, ng, nbeta,
           oW, ob, og, obeta, pW, pb):
    x = nodes.reshape(N, -1)
    senders = senders.astype(jnp.int32)
    receivers = receivers.astype(jnp.int32)
    h = _embed(x, Wn, bn.reshape(1, D))

    sA, sB = senders[:EH], senders[EH:]
    rA, rB = receivers[:EH], receivers[EH:]

    gA = _gather_half(h, sA, rA)
    gB = _gather_half(h, sB, rB)
    eA = _edge_first(edges[:EH], gA, We, be.reshape(1, D),
                     eW[0], eb[0], eg[0], ebeta[0])
    eB = _edge_first(edges[EH:], gB, We, be.reshape(1, D),
                     eW[0], eb[0], eg[0], ebeta[0])
    pA, cA = _scatter_half_counts_a(eA, rA)
    pB, cB = _scatter_half_counts_b(eB, rB)
    h = _node_mid(h, pA, pB, cA, cB, nW[0], nb[0], ng[0], nbeta[0])

    gA = _gather_half(h, sA, rA)
    gB = _gather_half(h, sB, rB)
    eA = _edge_rest(eA, gA, We, be.reshape(1, D),
                    eW[1], eb[1], eg[1], ebeta[1])
    eB = _edge_rest(eB, gB, We, be.reshape(1, D),
                    eW[1], eb[1], eg[1], ebeta[1])
    (pA,) = _scatter_half(eA, rA)
    (pB,) = _scatter_half(eB, rB)
    out = _node_last(h, pA, pB, cA, cB, nW[1], nb[1], ng[1], nbeta[1],
                     oW, ob.reshape(1, D), og.reshape(1, D),
                     obeta.reshape(1, D), pW, pb.reshape(1, PRED * COUT))
    return out.reshape(N, PRED, COUT)
